# baseline (speedup 1.0000x reference)
import functools

import jax
import jax.numpy as jnp
from jax.experimental import pallas as pl
from jax.experimental.pallas import tpu as pltpu

N = 10000
DIN = 128
DOUT = 128
BM = 400


def _gcn_kernel(adj_ref, x_full_ref, xbf_ref, wt_ref, eps_ref, out_ref):
    i = pl.program_id(0)
    neib = jnp.dot(adj_ref[...].astype(jnp.bfloat16), xbf_ref[...],
                   preferred_element_type=jnp.float32)
    x_tile = x_full_ref[pl.ds(i * BM, BM), :]
    h = neib + (1.0 + eps_ref[0, 0]) * x_tile
    out_ref[...] = jnp.dot(h, wt_ref[...],
                           preferred_element_type=jnp.float32)


@jax.jit
def kernel(x, adj, W, eps):
    wt = W.T
    eps2d = eps.reshape(1, 1)
    grid = (pl.cdiv(N, BM),)
    return pl.pallas_call(
        _gcn_kernel,
        grid=grid,
        in_specs=[
            pl.BlockSpec((BM, N), lambda i: (i, 0)),
            pl.BlockSpec((N, DIN), lambda i: (0, 0)),
            pl.BlockSpec((N, DIN), lambda i: (0, 0)),
            pl.BlockSpec((DIN, DOUT), lambda i: (0, 0)),
            pl.BlockSpec(memory_space=pltpu.SMEM),
        ],
        out_specs=pl.BlockSpec((BM, DOUT), lambda i: (i, 0)),
        out_shape=jax.ShapeDtypeStruct((N, DOUT), jnp.float32),
    )(adj, x, x.astype(jnp.bfloat16), wt, eps2d)


# speedup vs baseline: 1.0340x; 1.0340x over previous
import functools

import jax
import jax.numpy as jnp
from jax.experimental import pallas as pl
from jax.experimental.pallas import tpu as pltpu

N = 10000
DIN = 128
DOUT = 128
BM = 400


def _gcn_kernel(adj_ref, x_full_ref, wt_ref, eps_ref, out_ref):
    i = pl.program_id(0)
    neib = jnp.dot(adj_ref[...], x_full_ref[...],
                   preferred_element_type=jnp.float32)
    x_tile = x_full_ref[pl.ds(i * BM, BM), :]
    h = neib + (1.0 + eps_ref[0, 0]) * x_tile
    out_ref[...] = jnp.dot(h, wt_ref[...],
                           preferred_element_type=jnp.float32)


@jax.jit
def kernel(x, adj, W, eps):
    wt = W.T
    eps2d = eps.reshape(1, 1)
    grid = (pl.cdiv(N, BM),)
    return pl.pallas_call(
        _gcn_kernel,
        grid=grid,
        in_specs=[
            pl.BlockSpec((BM, N), lambda i: (i, 0)),
            pl.BlockSpec((N, DIN), lambda i: (0, 0)),
            pl.BlockSpec((DIN, DOUT), lambda i: (0, 0)),
            pl.BlockSpec(memory_space=pltpu.SMEM),
        ],
        out_specs=pl.BlockSpec((BM, DOUT), lambda i: (i, 0)),
        out_shape=jax.ShapeDtypeStruct((N, DOUT), jnp.float32),
    )(adj, x, wt, eps2d)
